# baseline (speedup 1.0000x reference)
import jax
import jax.numpy as jnp
from jax import lax
from jax.experimental import pallas as pl
from jax.experimental.pallas import tpu as pltpu

B, N = 512, 256
INPUT_DIM, COND_DIM, HIDDEN, LATENT, K = 3, 16, 256, 64, 1024
TCHUNK = 16
NVALID = N - 2
NCHUNK = (NVALID + TCHUNK - 1) // TCHUNK
NEG = float("-inf")

O_W1, O_W2, O_D1, O_D2, O_WZ, O_WC, O_WN, O_BIAS = (
    0, 256, 512, 768, 1024, 1088, 1104, 1152)
PK1_W = 1160
O_WL, O_BL, O_CBT = 0, 256, 384
PK2_W = 384 + K


def _fused_kernel(x_ref, w9t_ref, pk1_ref, pk2_ref, cb_ref, cn_ref, pk4_ref,
                  out_ref):
    f32 = jnp.float32
    run = jnp.full((HIDDEN, B), NEG, f32)
    for ch in range(NCHUNK):
        t0, t1 = ch * TCHUNK, min((ch + 1) * TCHUNK, NVALID)
        hp = jnp.dot(w9t_ref[:], x_ref[:, t0 * B:t1 * B],
                     preferred_element_type=f32)
        w = t1 - t0
        while w > 1:
            nw = w - w // 2
            hp = jnp.maximum(hp[:, :nw * B], hp[:, (w - nw) * B:w * B])
            w = nw
        run = jnp.maximum(run, hp)
    bias = lambda i: pk1_ref[:, O_BIAS + i:O_BIAS + i + 1]
    h = run + bias(5)
    h = jnp.maximum(jnp.dot(pk1_ref[:, O_W1:O_W1 + HIDDEN], h,
                            preferred_element_type=f32) + bias(0), 0.0)
    h = jnp.maximum(jnp.dot(pk1_ref[:, O_W2:O_W2 + HIDDEN], h,
                            preferred_element_type=f32) + bias(1), 0.0)
    z = (jnp.dot(pk2_ref[:, O_WL:O_WL + HIDDEN], h, preferred_element_type=f32)
         + pk2_ref[:, O_BL:O_BL + 1])
    cbn = jnp.sum(cb_ref[:] ** 2, axis=1, keepdims=True)
    dist = cbn - 2.0 * jnp.dot(cb_ref[:], z, preferred_element_type=f32)
    dmin = jnp.min(dist, axis=0, keepdims=True)
    ks = lax.broadcasted_iota(jnp.int32, (K, B), 0)
    idx = jnp.min(jnp.where(dist <= dmin, ks, K), axis=0)
    oh = jnp.where(ks == idx[None, :], 1.0, 0.0).astype(f32)
    zq = jnp.dot(pk2_ref[:, O_CBT:O_CBT + K], oh,
                 preferred_element_type=f32)
    part = (jnp.dot(pk1_ref[:, O_WC:O_WC + COND_DIM], cn_ref[:COND_DIM, :],
                    preferred_element_type=f32)
            + jnp.dot(pk1_ref[:, O_WN:O_WN + INPUT_DIM], cn_ref[COND_DIM:, :],
                      preferred_element_type=f32) + bias(2))
    d = jnp.maximum(jnp.dot(pk1_ref[:, O_WZ:O_WZ + LATENT], zq,
                            preferred_element_type=f32) + part, 0.0)
    d = jnp.maximum(jnp.dot(pk1_ref[:, O_D1:O_D1 + HIDDEN], d,
                            preferred_element_type=f32) + bias(3), 0.0)
    d = jnp.maximum(jnp.dot(pk1_ref[:, O_D2:O_D2 + HIDDEN], d,
                            preferred_element_type=f32) + bias(4), 0.0)
    out_ref[:, :] = (jnp.dot(pk4_ref[:, :HIDDEN], d, preferred_element_type=f32)
                     + pk4_ref[:, HIDDEN:HIDDEN + 1]).T


def kernel(x, c, noise, conv_w, conv_b, enc_h1_w, enc_h1_b, enc_h2_w, enc_h2_b,
           enc_lat_w, enc_lat_b, codebook, dec_in_w, dec_in_b, dec_h1_w, dec_h1_b,
           dec_h2_w, dec_h2_b, dec_out_w, dec_out_b):
    f32 = jnp.float32
    xt = jnp.transpose(x.astype(jnp.bfloat16), (2, 1, 0))
    x9 = jnp.stack([xt[cc, k:k + NVALID, :]
                    for k in range(3) for cc in range(3)])
    x9 = x9.reshape(9, NVALID * B)
    w9t = jnp.transpose(conv_w, (2, 1, 0)).reshape(9, HIDDEN).T.astype(jnp.bfloat16)
    col = lambda v: v[:, None].astype(f32)
    zpad = lambda w, n: jnp.concatenate(
        [w, jnp.zeros((HIDDEN, n - w.shape[1]), f32)], axis=1) if n > w.shape[1] else w
    pk1 = jnp.concatenate([
        enc_h1_w.T, enc_h2_w.T, dec_h1_w.T, dec_h2_w.T,
        dec_in_w[:LATENT].T,
        dec_in_w[LATENT:LATENT + COND_DIM].T,
        zpad(dec_in_w[LATENT + COND_DIM:].T, O_BIAS - O_WN),
        col(enc_h1_b), col(enc_h2_b), col(dec_in_b), col(dec_h1_b),
        col(dec_h2_b), col(conv_b),
        jnp.zeros((HIDDEN, PK1_W - O_BIAS - 6), f32)], axis=1)
    pk2 = jnp.concatenate([
        enc_lat_w.T, enc_lat_b[:, None].astype(f32),
        jnp.zeros((LATENT, O_CBT - O_BL - 1), f32), codebook.T], axis=1)
    cn = jnp.concatenate([c.T, noise.T], axis=0)
    pk4 = jnp.concatenate([dec_out_w.T, dec_out_b[:, None].astype(f32)], axis=1)

    out = pl.pallas_call(
        _fused_kernel,
        out_shape=jax.ShapeDtypeStruct((B, INPUT_DIM), f32),
    )(x9, w9t, pk1, pk2, codebook, cn, pk4)
    return out


# speedup vs baseline: 1.0025x; 1.0025x over previous
import jax
import jax.numpy as jnp
from jax import lax
from jax.experimental import pallas as pl
from jax.experimental.pallas import tpu as pltpu

B, N = 512, 256
INPUT_DIM, COND_DIM, HIDDEN, LATENT, K = 3, 16, 256, 64, 1024
TCHUNK = 8
NVALID = N - 2
NCHUNK = (NVALID + TCHUNK - 1) // TCHUNK
NEG = float("-inf")

O_W1, O_W2, O_D1, O_D2, O_WZ, O_WC, O_WN, O_BIAS = (
    0, 256, 512, 768, 1024, 1088, 1104, 1152)
PK1_W = 1160
O_WL, O_BL, O_CBT = 0, 256, 384
PK2_W = 384 + K


def _fused_kernel(x_ref, w9t_ref, pk1_ref, pk2_ref, cb_ref, cn_ref, pk4_ref,
                  out_ref):
    f32 = jnp.float32
    run = jnp.full((HIDDEN, B), NEG, f32)
    for ch in range(NCHUNK):
        t0, t1 = ch * TCHUNK, min((ch + 1) * TCHUNK, NVALID)
        hp = jnp.dot(w9t_ref[:], x_ref[:, t0 * B:t1 * B],
                     preferred_element_type=f32)
        w = t1 - t0
        while w > 1:
            nw = w - w // 2
            hp = jnp.maximum(hp[:, :nw * B], hp[:, (w - nw) * B:w * B])
            w = nw
        run = jnp.maximum(run, hp)
    bias = lambda i: pk1_ref[:, O_BIAS + i:O_BIAS + i + 1]
    h = run + bias(5)
    h = jnp.maximum(jnp.dot(pk1_ref[:, O_W1:O_W1 + HIDDEN], h,
                            preferred_element_type=f32) + bias(0), 0.0)
    h = jnp.maximum(jnp.dot(pk1_ref[:, O_W2:O_W2 + HIDDEN], h,
                            preferred_element_type=f32) + bias(1), 0.0)
    z = (jnp.dot(pk2_ref[:, O_WL:O_WL + HIDDEN], h, preferred_element_type=f32)
         + pk2_ref[:, O_BL:O_BL + 1])
    cbn = jnp.sum(cb_ref[:] ** 2, axis=1, keepdims=True)
    dist = cbn - 2.0 * jnp.dot(cb_ref[:], z, preferred_element_type=f32)
    dmin = jnp.min(dist, axis=0, keepdims=True)
    ks = lax.broadcasted_iota(jnp.int32, (K, B), 0)
    idx = jnp.min(jnp.where(dist <= dmin, ks, K), axis=0)
    oh = jnp.where(ks == idx[None, :], 1.0, 0.0).astype(f32)
    zq = jnp.dot(pk2_ref[:, O_CBT:O_CBT + K], oh,
                 preferred_element_type=f32)
    part = (jnp.dot(pk1_ref[:, O_WC:O_WC + COND_DIM], cn_ref[:COND_DIM, :],
                    preferred_element_type=f32)
            + jnp.dot(pk1_ref[:, O_WN:O_WN + INPUT_DIM], cn_ref[COND_DIM:, :],
                      preferred_element_type=f32) + bias(2))
    d = jnp.maximum(jnp.dot(pk1_ref[:, O_WZ:O_WZ + LATENT], zq,
                            preferred_element_type=f32) + part, 0.0)
    d = jnp.maximum(jnp.dot(pk1_ref[:, O_D1:O_D1 + HIDDEN], d,
                            preferred_element_type=f32) + bias(3), 0.0)
    d = jnp.maximum(jnp.dot(pk1_ref[:, O_D2:O_D2 + HIDDEN], d,
                            preferred_element_type=f32) + bias(4), 0.0)
    out_ref[:, :] = (jnp.dot(pk4_ref[:, :HIDDEN], d, preferred_element_type=f32)
                     + pk4_ref[:, HIDDEN:HIDDEN + 1]).T


def kernel(x, c, noise, conv_w, conv_b, enc_h1_w, enc_h1_b, enc_h2_w, enc_h2_b,
           enc_lat_w, enc_lat_b, codebook, dec_in_w, dec_in_b, dec_h1_w, dec_h1_b,
           dec_h2_w, dec_h2_b, dec_out_w, dec_out_b):
    f32 = jnp.float32
    xt = jnp.transpose(x.astype(jnp.bfloat16), (2, 1, 0))
    x9 = jnp.stack([xt[cc, k:k + NVALID, :]
                    for k in range(3) for cc in range(3)])
    x9 = x9.reshape(9, NVALID * B)
    w9t = jnp.transpose(conv_w, (2, 1, 0)).reshape(9, HIDDEN).T.astype(jnp.bfloat16)
    col = lambda v: v[:, None].astype(f32)
    zpad = lambda w, n: jnp.concatenate(
        [w, jnp.zeros((HIDDEN, n - w.shape[1]), f32)], axis=1) if n > w.shape[1] else w
    pk1 = jnp.concatenate([
        enc_h1_w.T, enc_h2_w.T, dec_h1_w.T, dec_h2_w.T,
        dec_in_w[:LATENT].T,
        dec_in_w[LATENT:LATENT + COND_DIM].T,
        zpad(dec_in_w[LATENT + COND_DIM:].T, O_BIAS - O_WN),
        col(enc_h1_b), col(enc_h2_b), col(dec_in_b), col(dec_h1_b),
        col(dec_h2_b), col(conv_b),
        jnp.zeros((HIDDEN, PK1_W - O_BIAS - 6), f32)], axis=1)
    pk2 = jnp.concatenate([
        enc_lat_w.T, enc_lat_b[:, None].astype(f32),
        jnp.zeros((LATENT, O_CBT - O_BL - 1), f32), codebook.T], axis=1)
    cn = jnp.concatenate([c.T, noise.T], axis=0)
    pk4 = jnp.concatenate([dec_out_w.T, dec_out_b[:, None].astype(f32)], axis=1)

    out = pl.pallas_call(
        _fused_kernel,
        out_shape=jax.ShapeDtypeStruct((B, INPUT_DIM), f32),
    )(x9, w9t, pk1, pk2, codebook, cn, pk4)
    return out
